# baseline (speedup 1.0000x reference)
import functools

import jax
import jax.numpy as jnp
from jax.experimental import pallas as pl
from jax.experimental.pallas import tpu as pltpu


def _hist_kernel(nbits, nhi, ec_ref, out_ref, acc_ref):
    step = pl.program_id(0)

    @pl.when(step == 0)
    def _():
        acc_ref[...] = jnp.zeros_like(acc_ref)

    ch = ec_ref.shape[0]
    for k in range(ch // 256):
        e = ec_ref[pl.ds(k * 256, 256)]
        d = e >> nbits
        hi = d >> 7
        lo = d & 127
        hoh = (hi == jax.lax.broadcasted_iota(jnp.int32, (256, nhi), 1)
               ).astype(jnp.bfloat16)
        loh = (lo == jax.lax.broadcasted_iota(jnp.int32, (256, 128), 1)
               ).astype(jnp.bfloat16)
        acc_ref[...] += jax.lax.dot_general(
            hoh, loh, dimension_numbers=(((0,), (0,)), ((), ())),
            preferred_element_type=jnp.float32)

    @pl.when(step == pl.num_programs(0) - 1)
    def _():
        hist = acc_ref[...]
        lane = jax.lax.broadcasted_iota(jnp.int32, (128, 128), 0)
        lane_t = jax.lax.broadcasted_iota(jnp.int32, (128, 128), 1)
        ut = (lane <= lane_t).astype(jnp.float32)
        xp = jax.lax.dot_general(
            hist, ut, dimension_numbers=(((1,), (0,)), ((), ())),
            preferred_element_type=jnp.float32,
            precision=jax.lax.Precision.HIGHEST)
        rs = jnp.sum(hist, axis=1, keepdims=True)
        row = jax.lax.broadcasted_iota(jnp.int32, (nhi, nhi), 0)
        row_t = jax.lax.broadcasted_iota(jnp.int32, (nhi, nhi), 1)
        lt = (row > row_t).astype(jnp.float32)
        ro = jax.lax.dot_general(
            lt, rs, dimension_numbers=(((1,), (0,)), ((), ())),
            preferred_element_type=jnp.float32,
            precision=jax.lax.Precision.HIGHEST)
        out_ref[...] = (xp + ro).astype(jnp.int32)


def _xw_kernel(x_ref, w_ref, dinv_ref, y_ref):
    acc = jax.lax.dot_general(
        x_ref[...], w_ref[...],
        dimension_numbers=(((1,), (1,)), ((), ())),
        preferred_element_type=jnp.float32)
    y_ref[...] = (dinv_ref[...] * acc).astype(y_ref.dtype)


def _agg_kernel(nbits, tb, ch, bs_ref, ec_ref, ecv_ref, yd2_ref, yd3_ref,
                dinv_ref, b_ref, h_ref, g_ref, acc_ref):
    blk = pl.program_id(0)
    base = blk * tb
    mask = (1 << nbits) - 1
    chsh = ch.bit_length() - 1
    lo = bs_ref[blk]
    hi = bs_ref[blk + 1]
    acc_ref[...] = yd2_ref[...].astype(jnp.float32)

    def chunk_body(c, carry):
        tbase = c * ch

        def gath(j, carry2):
            s = ec_ref[tbase + j] & mask
            g_ref[j] = yd3_ref[s]
            return carry2

        jax.lax.fori_loop(0, ch, gath, 0, unroll=False)
        dloc = (ecv_ref[c] >> nbits) - base
        doh = (jnp.broadcast_to(dloc, (tb, ch)) ==
               jax.lax.broadcasted_iota(jnp.int32, (tb, ch), 0)
               ).astype(jnp.bfloat16)
        g = g_ref[...].reshape(ch, g_ref.shape[-1])
        acc_ref[...] += jax.lax.dot_general(
            doh, g, dimension_numbers=(((1,), (0,)), ((), ())),
            preferred_element_type=jnp.float32)
        return carry

    jax.lax.fori_loop(lo >> chsh, (hi + ch - 1) >> chsh, chunk_body, 0)

    h = dinv_ref[...] * acc_ref[...] + b_ref[...]
    h_ref[...] = jnp.maximum(h, 0.0).astype(h_ref.dtype)


def _gram_kernel(hi_ref, hj_ref, o_ref):
    o_ref[...] = jax.lax.dot_general(
        hi_ref[...], hj_ref[...],
        dimension_numbers=(((1,), (1,)), ((), ())),
        preferred_element_type=jnp.float32)


def _pick(n, preferred):
    t = preferred
    while n % t:
        t //= 2
    return t


def kernel(x, edge_index, weight, bias):
    N, F = x.shape
    E = edge_index.shape[1]
    nbits = max(7, (N - 1).bit_length())
    nhi = N // 128

    src = edge_index[0].astype(jnp.int32)
    dst = edge_index[1].astype(jnp.int32)

    ec = jnp.sort((dst << nbits) | src)

    ch = 512
    e_pad = ((E + 2047) // 2048) * 2048
    ec_flat = jnp.pad(ec, (0, e_pad - E),
                      constant_values=jnp.int32(2**31 - 1))
    ec_pad = ec_flat.reshape(e_pad, 1)
    ec_rows = ec_flat.reshape(e_pad // ch, 1, ch)

    cdf = pl.pallas_call(
        functools.partial(_hist_kernel, nbits, nhi),
        out_shape=jax.ShapeDtypeStruct((nhi, 128), jnp.int32),
        grid=(e_pad // 2048,),
        in_specs=[pl.BlockSpec((2048, 1), lambda i: (i, 0))],
        out_specs=pl.BlockSpec((nhi, 128), lambda i: (0, 0)),
        scratch_shapes=[pltpu.VMEM((nhi, 128), jnp.float32)],
        compiler_params=pltpu.CompilerParams(
            dimension_semantics=("arbitrary",)),
    )(ec_pad)

    bounds = jnp.concatenate(
        [jnp.zeros((1,), jnp.int32), cdf.reshape(N)])
    deg = (bounds[1:] - bounds[:-1] + 1).astype(jnp.float32)
    dinv = jax.lax.rsqrt(deg).reshape(N, 1)

    xb = x.astype(jnp.bfloat16)
    wb = weight.astype(jnp.bfloat16)
    bf = bias.reshape(1, F).astype(jnp.float32)

    tm = _pick(N, 1024)
    yd = pl.pallas_call(
        _xw_kernel,
        out_shape=jax.ShapeDtypeStruct((N, F), jnp.bfloat16),
        grid=(N // tm,),
        in_specs=[
            pl.BlockSpec((tm, F), lambda i: (i, 0)),
            pl.BlockSpec((F, F), lambda i: (0, 0)),
            pl.BlockSpec((tm, 1), lambda i: (i, 0)),
        ],
        out_specs=pl.BlockSpec((tm, F), lambda i: (i, 0)),
        compiler_params=pltpu.CompilerParams(
            dimension_semantics=("parallel",)),
    )(xb, wb, dinv)

    yd3 = yd.reshape(N, 1, F)

    tb = _pick(N, 512)
    bs = bounds[::tb]
    h = pl.pallas_call(
        functools.partial(_agg_kernel, nbits, tb, ch),
        grid_spec=pltpu.PrefetchScalarGridSpec(
            num_scalar_prefetch=2,
            grid=(N // tb,),
            in_specs=[
                pl.BlockSpec((e_pad // ch, 1, ch),
                             lambda i, b_r, e_r: (0, 0, 0)),
                pl.BlockSpec((tb, F), lambda i, b_r, e_r: (i, 0)),
                pl.BlockSpec((N, 1, F), lambda i, b_r, e_r: (0, 0, 0)),
                pl.BlockSpec((tb, 1), lambda i, b_r, e_r: (i, 0)),
                pl.BlockSpec((1, F), lambda i, b_r, e_r: (0, 0)),
            ],
            out_specs=pl.BlockSpec((tb, F), lambda i, b_r, e_r: (i, 0)),
            scratch_shapes=[
                pltpu.VMEM((ch, 1, F), jnp.bfloat16),
                pltpu.VMEM((tb, F), jnp.float32),
            ],
        ),
        out_shape=jax.ShapeDtypeStruct((N, F), jnp.bfloat16),
        compiler_params=pltpu.CompilerParams(
            dimension_semantics=("parallel",),
            vmem_limit_bytes=56 * 1024 * 1024,
            disable_bounds_checks=True),
    )(bs, ec_flat, ec_rows, yd, yd3, dinv, bf)

    t2 = _pick(N, 1024)
    out = pl.pallas_call(
        _gram_kernel,
        out_shape=jax.ShapeDtypeStruct((N, N), jnp.float32),
        grid=(N // t2, N // t2),
        in_specs=[
            pl.BlockSpec((t2, F), lambda i, j: (i, 0)),
            pl.BlockSpec((t2, F), lambda i, j: (j, 0)),
        ],
        out_specs=pl.BlockSpec((t2, t2), lambda i, j: (i, j)),
        compiler_params=pltpu.CompilerParams(
            dimension_semantics=("parallel", "parallel")),
    )(h, h)

    return out


# speedup vs baseline: 1.2857x; 1.2857x over previous
import functools

import jax
import jax.numpy as jnp
from jax.experimental import pallas as pl
from jax.experimental.pallas import tpu as pltpu


def _hist_kernel(nbits, nhi, ec_ref, out_ref, acc_ref):
    step = pl.program_id(0)

    @pl.when(step == 0)
    def _():
        acc_ref[...] = jnp.zeros_like(acc_ref)

    ch = ec_ref.shape[0]
    for k in range(ch // 256):
        e = ec_ref[pl.ds(k * 256, 256)]
        d = e >> nbits
        hi = d >> 7
        lo = d & 127
        hoh = (hi == jax.lax.broadcasted_iota(jnp.int32, (256, nhi), 1)
               ).astype(jnp.bfloat16)
        loh = (lo == jax.lax.broadcasted_iota(jnp.int32, (256, 128), 1)
               ).astype(jnp.bfloat16)
        acc_ref[...] += jax.lax.dot_general(
            hoh, loh, dimension_numbers=(((0,), (0,)), ((), ())),
            preferred_element_type=jnp.float32)

    @pl.when(step == pl.num_programs(0) - 1)
    def _():
        hist = acc_ref[...]
        lane = jax.lax.broadcasted_iota(jnp.int32, (128, 128), 0)
        lane_t = jax.lax.broadcasted_iota(jnp.int32, (128, 128), 1)
        ut = (lane <= lane_t).astype(jnp.float32)
        xp = jax.lax.dot_general(
            hist, ut, dimension_numbers=(((1,), (0,)), ((), ())),
            preferred_element_type=jnp.float32,
            precision=jax.lax.Precision.HIGHEST)
        rs = jnp.sum(hist, axis=1, keepdims=True)
        row = jax.lax.broadcasted_iota(jnp.int32, (nhi, nhi), 0)
        row_t = jax.lax.broadcasted_iota(jnp.int32, (nhi, nhi), 1)
        lt = (row > row_t).astype(jnp.float32)
        ro = jax.lax.dot_general(
            lt, rs, dimension_numbers=(((1,), (0,)), ((), ())),
            preferred_element_type=jnp.float32,
            precision=jax.lax.Precision.HIGHEST)
        out_ref[...] = (xp + ro).astype(jnp.int32)


def _xw_kernel(x_ref, w_ref, dinv_ref, y_ref):
    acc = jax.lax.dot_general(
        x_ref[...], w_ref[...],
        dimension_numbers=(((1,), (1,)), ((), ())),
        preferred_element_type=jnp.float32)
    y_ref[...] = dinv_ref[...] * acc


def _gather_kernel(nbits, tb, bounds_ref, ec_ref, yd_ref, dinv_ref, b_ref,
                   h_ref, acc_ref):
    blk = pl.program_id(0)
    base = blk * tb
    mask = (1 << nbits) - 1

    def row_body(i, lo):
        v = base + i
        hi = bounds_ref[v + 1]

        def edge_body(t, carry):
            acc, row = carry
            nxt = yd_ref[ec_ref[t + 1] & mask]
            return (acc + row, nxt)

        row0 = yd_ref[ec_ref[lo] & mask]
        acc, _ = jax.lax.fori_loop(lo, hi, edge_body, (yd_ref[v], row0),
                                   unroll=False)
        acc_ref[i] = acc
        return hi

    jax.lax.fori_loop(0, tb, row_body, bounds_ref[base], unroll=False)
    h = dinv_ref[...] * acc_ref[...].reshape(h_ref.shape) + b_ref[...]
    h_ref[...] = jnp.maximum(h, 0.0).astype(h_ref.dtype)


def _gram_kernel(hi_ref, hj_ref, o_ref):
    o_ref[...] = jax.lax.dot_general(
        hi_ref[...], hj_ref[...],
        dimension_numbers=(((1,), (1,)), ((), ())),
        preferred_element_type=jnp.float32)


def _pick(n, preferred):
    t = preferred
    while n % t:
        t //= 2
    return t


def kernel(x, edge_index, weight, bias):
    N, F = x.shape
    E = edge_index.shape[1]
    nbits = max(7, (N - 1).bit_length())
    nhi = N // 128

    src = edge_index[0].astype(jnp.int32)
    dst = edge_index[1].astype(jnp.int32)

    ec = jnp.sort((dst << nbits) | src)

    ch = 2048
    e_pad = ((E + ch - 1) // ch) * ch
    ec_pad = jnp.pad(ec, (0, e_pad - E),
                     constant_values=jnp.int32(2**31 - 1)).reshape(e_pad, 1)

    cdf = pl.pallas_call(
        functools.partial(_hist_kernel, nbits, nhi),
        out_shape=jax.ShapeDtypeStruct((nhi, 128), jnp.int32),
        grid=(e_pad // ch,),
        in_specs=[pl.BlockSpec((ch, 1), lambda i: (i, 0))],
        out_specs=pl.BlockSpec((nhi, 128), lambda i: (0, 0)),
        scratch_shapes=[pltpu.VMEM((nhi, 128), jnp.float32)],
        compiler_params=pltpu.CompilerParams(
            dimension_semantics=("arbitrary",)),
    )(ec_pad)

    bounds = jnp.concatenate(
        [jnp.zeros((1,), jnp.int32), cdf.reshape(N)])
    deg = (bounds[1:] - bounds[:-1] + 1).astype(jnp.float32)
    dinv = jax.lax.rsqrt(deg)

    xb = x.astype(jnp.bfloat16)
    wb = weight.astype(jnp.bfloat16)
    bf = bias.reshape(1, F).astype(jnp.float32)

    tm = _pick(N, 1024)
    yd = pl.pallas_call(
        _xw_kernel,
        out_shape=jax.ShapeDtypeStruct((N, F), jnp.float32),
        grid=(N // tm,),
        in_specs=[
            pl.BlockSpec((tm, F), lambda i: (i, 0)),
            pl.BlockSpec((F, F), lambda i: (0, 0)),
            pl.BlockSpec((tm, 1), lambda i: (i, 0)),
        ],
        out_specs=pl.BlockSpec((tm, F), lambda i: (i, 0)),
        compiler_params=pltpu.CompilerParams(
            dimension_semantics=("parallel",)),
    )(xb, wb, dinv.reshape(N, 1))

    yd3 = yd.reshape(N, 1, F)

    tb = _pick(N, 512)
    h = pl.pallas_call(
        functools.partial(_gather_kernel, nbits, tb),
        grid_spec=pltpu.PrefetchScalarGridSpec(
            num_scalar_prefetch=2,
            grid=(N // tb,),
            in_specs=[
                pl.BlockSpec((N, 1, F), lambda i, b_r, e_r: (0, 0, 0)),
                pl.BlockSpec((tb, 1), lambda i, b_r, e_r: (i, 0)),
                pl.BlockSpec((1, F), lambda i, b_r, e_r: (0, 0)),
            ],
            out_specs=pl.BlockSpec((tb, F), lambda i, b_r, e_r: (i, 0)),
            scratch_shapes=[pltpu.VMEM((tb, 1, F), jnp.float32)],
        ),
        out_shape=jax.ShapeDtypeStruct((N, F), jnp.bfloat16),
        compiler_params=pltpu.CompilerParams(
            dimension_semantics=("parallel",),
            vmem_limit_bytes=56 * 1024 * 1024,
            disable_bounds_checks=True),
    )(bounds, ec_pad.reshape(e_pad), yd3, dinv.reshape(N, 1), bf)

    t2 = _pick(N, 1024)
    out = pl.pallas_call(
        _gram_kernel,
        out_shape=jax.ShapeDtypeStruct((N, N), jnp.float32),
        grid=(N // t2, N // t2),
        in_specs=[
            pl.BlockSpec((t2, F), lambda i, j: (i, 0)),
            pl.BlockSpec((t2, F), lambda i, j: (j, 0)),
        ],
        out_specs=pl.BlockSpec((t2, t2), lambda i, j: (i, j)),
        compiler_params=pltpu.CompilerParams(
            dimension_semantics=("parallel", "parallel")),
    )(h, h)

    return out


# speedup vs baseline: 1.4073x; 1.0945x over previous
import functools

import jax
import jax.numpy as jnp
from jax.experimental import pallas as pl
from jax.experimental.pallas import tpu as pltpu


def _hist_kernel(nbits, nhi, ec_ref, out_ref, acc_ref):
    step = pl.program_id(0)

    @pl.when(step == 0)
    def _():
        acc_ref[...] = jnp.zeros_like(acc_ref)

    ch = ec_ref.shape[0]
    for k in range(ch // 256):
        e = ec_ref[pl.ds(k * 256, 256)]
        d = e >> nbits
        hi = d >> 7
        lo = d & 127
        hoh = (hi == jax.lax.broadcasted_iota(jnp.int32, (256, nhi), 1)
               ).astype(jnp.bfloat16)
        loh = (lo == jax.lax.broadcasted_iota(jnp.int32, (256, 128), 1)
               ).astype(jnp.bfloat16)
        acc_ref[...] += jax.lax.dot_general(
            hoh, loh, dimension_numbers=(((0,), (0,)), ((), ())),
            preferred_element_type=jnp.float32)

    @pl.when(step == pl.num_programs(0) - 1)
    def _():
        hist = acc_ref[...]
        lane = jax.lax.broadcasted_iota(jnp.int32, (128, 128), 0)
        lane_t = jax.lax.broadcasted_iota(jnp.int32, (128, 128), 1)
        ut = (lane <= lane_t).astype(jnp.float32)
        xp = jax.lax.dot_general(
            hist, ut, dimension_numbers=(((1,), (0,)), ((), ())),
            preferred_element_type=jnp.float32,
            precision=jax.lax.Precision.HIGHEST)
        rs = jnp.sum(hist, axis=1, keepdims=True)
        row = jax.lax.broadcasted_iota(jnp.int32, (nhi, nhi), 0)
        row_t = jax.lax.broadcasted_iota(jnp.int32, (nhi, nhi), 1)
        lt = (row > row_t).astype(jnp.float32)
        ro = jax.lax.dot_general(
            lt, rs, dimension_numbers=(((1,), (0,)), ((), ())),
            preferred_element_type=jnp.float32,
            precision=jax.lax.Precision.HIGHEST)
        out_ref[...] = (xp + ro).astype(jnp.int32)


def _xw_kernel(x_ref, w_ref, dinv_ref, y_ref):
    acc = jax.lax.dot_general(
        x_ref[...], w_ref[...],
        dimension_numbers=(((1,), (1,)), ((), ())),
        preferred_element_type=jnp.float32)
    y_ref[...] = dinv_ref[...] * acc


def _gather_kernel(nbits, tb, bounds_ref, ec_ref, yd_ref, dinv_ref, b_ref,
                   h_ref, acc_ref):
    blk = pl.program_id(0)
    base = blk * tb
    mask = (1 << nbits) - 1

    def row_body(i, lo):
        v = base + i
        hi = bounds_ref[v + 1]

        def edge_body(k, carry):
            acc, ra, rb = carry
            t = lo + 2 * k
            na = yd_ref[ec_ref[t + 2] & mask]
            nb = yd_ref[ec_ref[t + 3] & mask]
            acc = acc + ra
            acc = acc + jnp.where(t + 1 < hi, rb, 0.0)
            return (acc, na, nb)

        ra0 = yd_ref[ec_ref[lo] & mask]
        rb0 = yd_ref[ec_ref[lo + 1] & mask]
        acc, _, _ = jax.lax.fori_loop(0, (hi - lo + 1) >> 1, edge_body,
                                      (yd_ref[v], ra0, rb0), unroll=False)
        acc_ref[i] = acc
        return hi

    jax.lax.fori_loop(0, tb, row_body, bounds_ref[base], unroll=False)
    h = dinv_ref[...] * acc_ref[...].reshape(h_ref.shape) + b_ref[...]
    h_ref[...] = jnp.maximum(h, 0.0).astype(h_ref.dtype)


def _gram_kernel(hi_ref, hj_ref, o_ref):
    o_ref[...] = jax.lax.dot_general(
        hi_ref[...], hj_ref[...],
        dimension_numbers=(((1,), (1,)), ((), ())),
        preferred_element_type=jnp.float32)


def _pick(n, preferred):
    t = preferred
    while n % t:
        t //= 2
    return t


def kernel(x, edge_index, weight, bias):
    N, F = x.shape
    E = edge_index.shape[1]
    nbits = max(7, (N - 1).bit_length())
    nhi = N // 128

    src = edge_index[0].astype(jnp.int32)
    dst = edge_index[1].astype(jnp.int32)

    ec = jnp.sort((dst << nbits) | src)

    ch = 2048
    e_pad = ((E + ch - 1) // ch) * ch
    if e_pad - E < 4:
        e_pad += ch
    ec_pad = jnp.pad(ec, (0, e_pad - E),
                     constant_values=jnp.int32(2**31 - 1)).reshape(e_pad, 1)

    cdf = pl.pallas_call(
        functools.partial(_hist_kernel, nbits, nhi),
        out_shape=jax.ShapeDtypeStruct((nhi, 128), jnp.int32),
        grid=(e_pad // ch,),
        in_specs=[pl.BlockSpec((ch, 1), lambda i: (i, 0))],
        out_specs=pl.BlockSpec((nhi, 128), lambda i: (0, 0)),
        scratch_shapes=[pltpu.VMEM((nhi, 128), jnp.float32)],
        compiler_params=pltpu.CompilerParams(
            dimension_semantics=("arbitrary",)),
    )(ec_pad)

    bounds = jnp.concatenate(
        [jnp.zeros((1,), jnp.int32), cdf.reshape(N)])
    deg = (bounds[1:] - bounds[:-1] + 1).astype(jnp.float32)
    dinv = jax.lax.rsqrt(deg)

    xb = x.astype(jnp.bfloat16)
    wb = weight.astype(jnp.bfloat16)
    bf = bias.reshape(1, F).astype(jnp.float32)

    tm = _pick(N, 1024)
    yd = pl.pallas_call(
        _xw_kernel,
        out_shape=jax.ShapeDtypeStruct((N, F), jnp.float32),
        grid=(N // tm,),
        in_specs=[
            pl.BlockSpec((tm, F), lambda i: (i, 0)),
            pl.BlockSpec((F, F), lambda i: (0, 0)),
            pl.BlockSpec((tm, 1), lambda i: (i, 0)),
        ],
        out_specs=pl.BlockSpec((tm, F), lambda i: (i, 0)),
        compiler_params=pltpu.CompilerParams(
            dimension_semantics=("parallel",)),
    )(xb, wb, dinv.reshape(N, 1))

    yd3 = yd.reshape(N, 1, F)

    tb = _pick(N, 512)
    h = pl.pallas_call(
        functools.partial(_gather_kernel, nbits, tb),
        grid_spec=pltpu.PrefetchScalarGridSpec(
            num_scalar_prefetch=2,
            grid=(N // tb,),
            in_specs=[
                pl.BlockSpec((N, 1, F), lambda i, b_r, e_r: (0, 0, 0)),
                pl.BlockSpec((tb, 1), lambda i, b_r, e_r: (i, 0)),
                pl.BlockSpec((1, F), lambda i, b_r, e_r: (0, 0)),
            ],
            out_specs=pl.BlockSpec((tb, F), lambda i, b_r, e_r: (i, 0)),
            scratch_shapes=[pltpu.VMEM((tb, 1, F), jnp.float32)],
        ),
        out_shape=jax.ShapeDtypeStruct((N, F), jnp.bfloat16),
        compiler_params=pltpu.CompilerParams(
            dimension_semantics=("parallel",),
            vmem_limit_bytes=56 * 1024 * 1024,
            disable_bounds_checks=True),
    )(bounds, ec_pad.reshape(e_pad), yd3, dinv.reshape(N, 1), bf)

    t2 = _pick(N, 1024)
    out = pl.pallas_call(
        _gram_kernel,
        out_shape=jax.ShapeDtypeStruct((N, N), jnp.float32),
        grid=(N // t2, N // t2),
        in_specs=[
            pl.BlockSpec((t2, F), lambda i, j: (i, 0)),
            pl.BlockSpec((t2, F), lambda i, j: (j, 0)),
        ],
        out_specs=pl.BlockSpec((t2, t2), lambda i, j: (i, j)),
        compiler_params=pltpu.CompilerParams(
            dimension_semantics=("parallel", "parallel")),
    )(h, h)

    return out
